# baseline (speedup 1.0000x reference)
import functools

import jax
import jax.numpy as jnp
from jax import lax
from jax.experimental import pallas as pl
from jax.experimental.pallas import tpu as pltpu
from jax.experimental.pallas import tpu_sc as plsc

_MAX_RECENCY = 1000
_R_SIZE = 64
_BATCH = 16384

_NC = 2
_NS = 16
_L = 16
_NW = _NC * _NS
_BPW = _BATCH // _NW
_CHUNK = 128
_NCHUNK = _BPW // _CHUNK


def _make_kernel():
  mesh = plsc.VectorSubcoreMesh(core_axis_name="c", subcore_axis_name="s")

  @functools.partial(
      pl.kernel,
      mesh=mesh,
      out_type=jax.ShapeDtypeStruct((_BATCH, _R_SIZE), jnp.float32),
      scratch_types=[
          pltpu.VMEM((_BPW,), jnp.int32),
          pltpu.VMEM((_BPW, _R_SIZE), jnp.float32),
          [pltpu.SemaphoreType.DMA] * _NCHUNK,
          [pltpu.SemaphoreType.DMA] * _NCHUNK,
          pltpu.SemaphoreType.DMA,
      ],
      compiler_params=pltpu.CompilerParams(use_tc_tiling_on_sc=False),
  )
  def emb(idx_hbm, table_hbm, out_hbm, idx_v, rows_v, isems, gsems, ssem):
    wid = lax.axis_index("s") * _NC + lax.axis_index("c")
    base = wid * _BPW
    iloads = []
    for j in range(_NCHUNK):
      c = pl.ds(j * _CHUNK, _CHUNK)
      iloads.append(
          pltpu.async_copy(
              idx_hbm.at[pl.ds(base + j * _CHUNK, _CHUNK)], idx_v.at[c],
              isems[j]))
    gathers = []
    for j in range(_NCHUNK):
      iloads[j].wait()
      for i in range(_CHUNK // _L):
        sl = pl.ds(j * _CHUNK + i * _L, _L)
        idx_v[sl] = jnp.minimum(idx_v[sl], _MAX_RECENCY - 1)
      c = pl.ds(j * _CHUNK, _CHUNK)
      gathers.append(
          pltpu.async_copy(table_hbm.at[idx_v.at[c]], rows_v.at[c], gsems[j]))
    stores = []
    for j in range(_NCHUNK):
      c = pl.ds(j * _CHUNK, _CHUNK)
      gathers[j].wait()
      stores.append(
          pltpu.async_copy(
              rows_v.at[c], out_hbm.at[pl.ds(base + j * _CHUNK, _CHUNK)],
              ssem))
    for st in stores:
      st.wait()

  return emb


_emb = _make_kernel()


def kernel(recency, table):
  return _emb(recency, table)


# speedup vs baseline: 1.1022x; 1.1022x over previous
import functools

import jax
import jax.numpy as jnp
from jax import lax
from jax.experimental import pallas as pl
from jax.experimental.pallas import tpu as pltpu
from jax.experimental.pallas import tpu_sc as plsc

_MAX_RECENCY = 1000
_R_SIZE = 64
_BATCH = 16384

_NC = 2
_NS = 16
_L = 16
_NW = _NC * _NS
_BPW = _BATCH // _NW
_CHUNK = 128
_NCHUNK = _BPW // _CHUNK


def _make_kernel():
  mesh = plsc.VectorSubcoreMesh(core_axis_name="c", subcore_axis_name="s")

  @functools.partial(
      pl.kernel,
      mesh=mesh,
      out_type=jax.ShapeDtypeStruct((_BATCH, _R_SIZE), jnp.float32),
      scratch_types=[
          pltpu.VMEM((_BPW,), jnp.int32),
          pltpu.VMEM((_BPW, _R_SIZE), jnp.float32),
          pltpu.VMEM_SHARED((_MAX_RECENCY, _R_SIZE), jnp.float32),
          pltpu.SemaphoreType.DMA,
          [pltpu.SemaphoreType.DMA] * _NCHUNK,
          pltpu.SemaphoreType.DMA,
      ],
      compiler_params=pltpu.CompilerParams(use_tc_tiling_on_sc=False),
  )
  def emb(idx_hbm, table_hbm, out_hbm, idx_v, rows_v, table_s, isem, gsems,
          ssem):
    sid = lax.axis_index("s")
    wid = sid * _NC + lax.axis_index("c")
    base = wid * _BPW
    iload = pltpu.async_copy(idx_hbm.at[pl.ds(base, _BPW)], idx_v, isem)

    @pl.when(sid == 0)
    def _stage_table():
      pltpu.sync_copy(table_hbm, table_s)

    iload.wait()
    for i in range(_BPW // _L):
      sl = pl.ds(i * _L, _L)
      idx_v[sl] = jnp.minimum(idx_v[sl], _MAX_RECENCY - 1)
    plsc.subcore_barrier()
    gathers = []
    for j in range(_NCHUNK):
      c = pl.ds(j * _CHUNK, _CHUNK)
      gathers.append(
          pltpu.async_copy(table_s.at[idx_v.at[c]], rows_v.at[c], gsems[j]))
    stores = []
    for j in range(_NCHUNK):
      c = pl.ds(j * _CHUNK, _CHUNK)
      gathers[j].wait()
      stores.append(
          pltpu.async_copy(
              rows_v.at[c], out_hbm.at[pl.ds(base + j * _CHUNK, _CHUNK)],
              ssem))
    for st in stores:
      st.wait()

  return emb


_emb = _make_kernel()


def kernel(recency, table):
  return _emb(recency, table)


# speedup vs baseline: 1.1053x; 1.0028x over previous
import functools

import jax
import jax.numpy as jnp
from jax import lax
from jax.experimental import pallas as pl
from jax.experimental.pallas import tpu as pltpu
from jax.experimental.pallas import tpu_sc as plsc

_MAX_RECENCY = 1000
_R_SIZE = 64
_BATCH = 16384

_NC = 2
_NS = 16
_L = 16
_NW = _NC * _NS
_BPW = _BATCH // _NW
_CHUNK = 128
_NCHUNK = _BPW // _CHUNK
_STAGERS = 8
_ROWS_PER_STAGER = _MAX_RECENCY // _STAGERS


def _make_kernel():
  mesh = plsc.VectorSubcoreMesh(core_axis_name="c", subcore_axis_name="s")

  @functools.partial(
      pl.kernel,
      mesh=mesh,
      out_type=jax.ShapeDtypeStruct((_BATCH, _R_SIZE), jnp.float32),
      scratch_types=[
          pltpu.VMEM((_BPW,), jnp.int32),
          pltpu.VMEM((_BPW, _R_SIZE), jnp.float32),
          pltpu.VMEM_SHARED((_MAX_RECENCY, _R_SIZE), jnp.float32),
          [pltpu.SemaphoreType.DMA] * _NCHUNK,
          [pltpu.SemaphoreType.DMA] * _NCHUNK,
          pltpu.SemaphoreType.DMA,
      ],
      compiler_params=pltpu.CompilerParams(use_tc_tiling_on_sc=False),
  )
  def emb(idx_hbm, table_hbm, out_hbm, idx_v, rows_v, table_s, isems, gsems,
          ssem):
    sid = lax.axis_index("s")
    wid = sid * _NC + lax.axis_index("c")
    base = wid * _BPW
    iloads = []
    for j in range(_NCHUNK):
      c = pl.ds(j * _CHUNK, _CHUNK)
      iloads.append(
          pltpu.async_copy(
              idx_hbm.at[pl.ds(base + j * _CHUNK, _CHUNK)], idx_v.at[c],
              isems[j]))

    @pl.when(sid < _STAGERS)
    def _stage_table():
      r = pl.ds(sid * _ROWS_PER_STAGER, _ROWS_PER_STAGER)
      pltpu.sync_copy(table_hbm.at[r], table_s.at[r])

    plsc.subcore_barrier()
    gathers = []
    for j in range(_NCHUNK):
      iloads[j].wait()
      for i in range(_CHUNK // _L):
        sl = pl.ds(j * _CHUNK + i * _L, _L)
        idx_v[sl] = jnp.minimum(idx_v[sl], _MAX_RECENCY - 1)
      c = pl.ds(j * _CHUNK, _CHUNK)
      gathers.append(
          pltpu.async_copy(table_s.at[idx_v.at[c]], rows_v.at[c], gsems[j]))
    stores = []
    for j in range(_NCHUNK):
      c = pl.ds(j * _CHUNK, _CHUNK)
      gathers[j].wait()
      stores.append(
          pltpu.async_copy(
              rows_v.at[c], out_hbm.at[pl.ds(base + j * _CHUNK, _CHUNK)],
              ssem))
    for st in stores:
      st.wait()

  return emb


_emb = _make_kernel()


def kernel(recency, table):
  return _emb(recency, table)
